# baseline (speedup 1.0000x reference)
import functools
import jax
import jax.numpy as jnp
from jax import lax
from jax.experimental import pallas as pl
from jax.experimental.pallas import tpu as pltpu
from jax.experimental.pallas import tpu_sc as plsc

_C = 32
_U = 8
_NW = 32


def _make_sc_call(b, s, d):
    rows_w = s // _NW
    nchunks = rows_w // _C
    n_items = nchunks * b
    nvec = d // 16
    mesh = plsc.VectorSubcoreMesh(core_axis_name="c", subcore_axis_name="s")

    @functools.partial(
        pl.kernel,
        mesh=mesh,
        out_type=jax.ShapeDtypeStruct((b, s, d), jnp.float32),
        scratch_types=[
            pltpu.VMEM((_C, d), jnp.float32),
            pltpu.VMEM((_C, d), jnp.float32),
            pltpu.VMEM((_C, d), jnp.float32),
            pltpu.SemaphoreType.DMA,
            pltpu.SemaphoreType.DMA,
            pltpu.SemaphoreType.DMA,
            pltpu.SemaphoreType.DMA,
        ],
    )
    def k(in_hbm, emb_hbm, out_hbm, emb_v, io0, io1, isem0, isem1, osem0, osem1):
        wid = lax.axis_index("s") * 2 + lax.axis_index("c")
        base = wid * rows_w
        io = (io0, io1)
        isem = (isem0, isem1)
        osem = (osem0, osem1)

        def add_chunk(buf):
            def row_body(r, _):
                def vec_body(v, __):
                    for u in range(_U):
                        sl = pl.ds((v * _U + u) * 16, 16)
                        buf[r, sl] = buf[r, sl] + emb_v[r, sl]
                    return __
                return lax.fori_loop(0, nvec // _U, vec_body, _)
            lax.fori_loop(0, _C, row_body, 0)

        def item_src(t):
            i, bb = divmod(t, b)
            return (bb, base + i * _C)

        bb0, r0 = item_src(0)
        in_h = [None, None]
        out_h = [None, None]
        in_h[0] = pltpu.async_copy(in_hbm.at[bb0, pl.ds(r0, _C)], io0, isem0)

        for t in range(n_items):
            p = t % 2
            i, bb = divmod(t, b)
            if bb == 0:
                pltpu.sync_copy(emb_hbm.at[pl.ds(base + i * _C, _C)], emb_v)
            in_h[p].wait()
            add_chunk(io[p])
            out_h[p] = pltpu.async_copy(
                io[p], out_hbm.at[bb, pl.ds(base + i * _C, _C)], osem[p]
            )
            if t + 1 < n_items:
                q = (t + 1) % 2
                if out_h[q] is not None:
                    out_h[q].wait()
                bb1, r1 = item_src(t + 1)
                in_h[q] = pltpu.async_copy(
                    in_hbm.at[bb1, pl.ds(r1, _C)], io[q], isem[q]
                )

        out_h[(n_items - 1) % 2].wait()
        if out_h[n_items % 2] is not None:
            out_h[n_items % 2].wait()

    return k


def kernel(inputs, embeddings):
    b, s, d = inputs.shape
    return _make_sc_call(b, s, d)(inputs, embeddings[:s])


# speedup vs baseline: 5.2342x; 5.2342x over previous
import jax
import jax.numpy as jnp
from jax import lax
from jax.experimental import pallas as pl
from jax.experimental.pallas import tpu as pltpu

_RB = 128
_NB = 4


def _make(b, s, d):
    nsteps = s // _RB
    ngroups = nsteps // _NB

    def body(in_hbm, emb_hbm, out_hbm, emb_v, ibufs, obufs, esem, isems, osems):
        pltpu.make_async_copy(emb_hbm, emb_v, esem).start()
        for j in range(_NB):
            for bb in range(b):
                pltpu.make_async_copy(
                    in_hbm.at[bb, pl.ds(j * _RB, _RB)], ibufs.at[j, bb], isems.at[j]
                ).start()
        pltpu.make_async_copy(emb_hbm, emb_v, esem).wait()

        def group(g, _):
            for j in range(_NB):
                t = g * _NB + j
                row0 = t * _RB
                for bb in range(b):
                    pltpu.make_async_copy(
                        in_hbm.at[bb, pl.ds(row0, _RB)], ibufs.at[j, bb], isems.at[j]
                    ).wait()

                @pl.when(t >= _NB)
                def _wait_prev_out():
                    for bb in range(b):
                        pltpu.make_async_copy(
                            obufs.at[j, bb],
                            out_hbm.at[bb, pl.ds((t - _NB) * _RB, _RB)],
                            osems.at[j],
                        ).wait()

                obufs[j] = ibufs[j] + emb_v[pl.ds(row0, _RB), :][jnp.newaxis]
                for bb in range(b):
                    pltpu.make_async_copy(
                        obufs.at[j, bb], out_hbm.at[bb, pl.ds(row0, _RB)], osems.at[j]
                    ).start()

                @pl.when(t + _NB < nsteps)
                def _prefetch_in():
                    for bb in range(b):
                        pltpu.make_async_copy(
                            in_hbm.at[bb, pl.ds((t + _NB) * _RB, _RB)],
                            ibufs.at[j, bb],
                            isems.at[j],
                        ).start()

            return _

        lax.fori_loop(0, ngroups, group, 0)
        for j in range(_NB):
            t = (ngroups - 1) * _NB + j
            for bb in range(b):
                pltpu.make_async_copy(
                    obufs.at[j, bb], out_hbm.at[bb, pl.ds(t * _RB, _RB)], osems.at[j]
                ).wait()

    return pl.pallas_call(
        body,
        in_specs=[
            pl.BlockSpec(memory_space=pl.ANY),
            pl.BlockSpec(memory_space=pl.ANY),
        ],
        out_specs=pl.BlockSpec(memory_space=pl.ANY),
        out_shape=jax.ShapeDtypeStruct((b, s, d), jnp.float32),
        scratch_shapes=[
            pltpu.VMEM((s, d), jnp.float32),
            pltpu.VMEM((_NB, b, _RB, d), jnp.float32),
            pltpu.VMEM((_NB, b, _RB, d), jnp.float32),
            pltpu.SemaphoreType.DMA,
            pltpu.SemaphoreType.DMA((_NB,)),
            pltpu.SemaphoreType.DMA((_NB,)),
        ],
    )


def kernel(inputs, embeddings):
    b, s, d = inputs.shape
    return _make(b, s, d)(inputs, embeddings[:s])
